# baseline (speedup 1.0000x reference)
import functools

import jax
import jax.numpy as jnp
from jax import lax
from jax.experimental import pallas as pl
from jax.experimental.pallas import tpu as pltpu
from jax.experimental.pallas import tpu_sc as plsc

N_FACTORS = 32
BATCH = 16384

NUM_CORES = 2
NUM_SUBCORES = 16
LANES = 16
NUM_WORKERS = NUM_CORES * NUM_SUBCORES
B_PER_W = BATCH // NUM_WORKERS
IDX_CHUNK = 128
N_CHUNKS = B_PER_W // IDX_CHUNK
N_GROUPS = B_PER_W // LANES

_mesh = plsc.VectorSubcoreMesh(
    core_axis_name="c", subcore_axis_name="s",
    num_cores=NUM_CORES, num_subcores=NUM_SUBCORES,
)


@functools.partial(
    pl.kernel,
    out_type=jax.ShapeDtypeStruct((BATCH,), jnp.float32),
    mesh=_mesh,
    compiler_params=pltpu.CompilerParams(
        needs_layout_passes=False, use_tc_tiling_on_sc=False),
    scratch_types=dict(
        uidx=pltpu.VMEM((N_CHUNKS, IDX_CHUNK), jnp.int32),
        midx=pltpu.VMEM((N_CHUNKS, IDX_CHUNK), jnp.int32),
        urows=pltpu.VMEM((B_PER_W, N_FACTORS), jnp.float32),
        mrows=pltpu.VMEM((B_PER_W, N_FACTORS), jnp.float32),
        out_v=pltpu.VMEM((B_PER_W,), jnp.float32),
        sem=pltpu.SemaphoreType.DMA,
    ),
)
def _sc_body(user_ids, movie_ids, user_factors, movie_factors, out_hbm,
             uidx, midx, urows, mrows, out_v, sem):
    wid = lax.axis_index("s") * NUM_CORES + lax.axis_index("c")
    base = wid * B_PER_W

    for c in range(N_CHUNKS):
        off = base + c * IDX_CHUNK
        pltpu.sync_copy(user_ids.at[pl.ds(off, IDX_CHUNK)], uidx.at[c])
        pltpu.sync_copy(movie_ids.at[pl.ds(off, IDX_CHUNK)], midx.at[c])

    copies = []
    for c in range(N_CHUNKS):
        dst = urows.at[pl.ds(c * IDX_CHUNK, IDX_CHUNK), :]
        copies.append(pltpu.async_copy(user_factors.at[uidx.at[c]], dst, sem))
        dst = mrows.at[pl.ds(c * IDX_CHUNK, IDX_CHUNK), :]
        copies.append(pltpu.async_copy(movie_factors.at[midx.at[c]], dst, sem))
    for cp in copies:
        cp.wait()

    lane = lax.broadcasted_iota(jnp.int32, (LANES,), 0)
    last_lane = lane == (LANES - 1)

    def row_body(r, _):
        a_lo = urows[r, pl.ds(0, LANES)]
        a_hi = urows[r, pl.ds(LANES, LANES)]
        b_lo = mrows[r, pl.ds(0, LANES)]
        b_hi = mrows[r, pl.ds(LANES, LANES)]
        p = a_lo * b_lo + a_hi * b_hi
        s = plsc.cumsum(p)
        plsc.store_scatter(out_v, [jnp.full((LANES,), r, jnp.int32)], s,
                           mask=last_lane)
        return 0

    lax.fori_loop(0, B_PER_W, row_body, 0)

    pltpu.sync_copy(out_v, out_hbm.at[pl.ds(base, B_PER_W)])


def kernel(user_ids, movie_ids, user_factors, movie_factors):
    out = _sc_body(
        user_ids.astype(jnp.int32),
        movie_ids.astype(jnp.int32),
        user_factors,
        movie_factors,
    )
    return out.reshape(-1, 1)
